# baseline (speedup 1.0000x reference)
import functools

import jax
import jax.numpy as jnp
from jax.experimental import pallas as pl
from jax.experimental.pallas import tpu as pltpu


def _mm_body(adj_ref, whi_ref, wlo_ref, out_ref, *, bk: int):
    j = pl.program_id(1)

    @pl.when(j == 0)
    def _():
        out_ref[...] = jnp.zeros_like(out_ref)

    a = adj_ref[...]
    a_hi = a.astype(jnp.bfloat16)
    a_lo = (a - a_hi.astype(jnp.float32)).astype(jnp.bfloat16)
    w_hi = whi_ref[pl.ds(j * bk, bk), :]
    w_lo = wlo_ref[pl.ds(j * bk, bk), :]
    acc = jnp.dot(a_hi, w_hi, preferred_element_type=jnp.float32)
    acc += jnp.dot(a_hi, w_lo, preferred_element_type=jnp.float32)
    acc += jnp.dot(a_lo, w_hi, preferred_element_type=jnp.float32)
    out_ref[...] += acc


def kernel(adj, weight):
    m, k = adj.shape
    k2, n = weight.shape
    assert k == k2
    w_hi = weight.astype(jnp.bfloat16)
    w_lo = (weight - w_hi.astype(jnp.float32)).astype(jnp.bfloat16)
    bm = 512
    bk = 2048
    grid = (m // bm, k // bk)
    return pl.pallas_call(
        functools.partial(_mm_body, bk=bk),
        grid=grid,
        in_specs=[
            pl.BlockSpec((bm, bk), lambda i, j: (i, j)),
            pl.BlockSpec((k2, n), lambda i, j: (0, 0)),
            pl.BlockSpec((k2, n), lambda i, j: (0, 0)),
        ],
        out_specs=pl.BlockSpec((bm, n), lambda i, j: (i, 0)),
        out_shape=jax.ShapeDtypeStruct((m, n), jnp.float32),
        compiler_params=pltpu.CompilerParams(
            dimension_semantics=("parallel", "arbitrary"),
        ),
    )(adj, w_hi, w_lo)


# speedup vs baseline: 1.6516x; 1.6516x over previous
import functools

import jax
import jax.numpy as jnp
from jax.experimental import pallas as pl
from jax.experimental.pallas import tpu as pltpu


def _mm_body(adj_ref, w_ref, out_ref):
    out_ref[...] = jnp.dot(
        adj_ref[...], w_ref[...], preferred_element_type=jnp.float32
    )


def kernel(adj, weight):
    m, k = adj.shape
    k2, n = weight.shape
    assert k == k2
    bm = 128
    grid = (m // bm,)
    return pl.pallas_call(
        _mm_body,
        grid=grid,
        in_specs=[
            pl.BlockSpec((bm, k), lambda i: (i, 0)),
            pl.BlockSpec((k2, n), lambda i: (0, 0)),
        ],
        out_specs=pl.BlockSpec((bm, n), lambda i: (i, 0)),
        out_shape=jax.ShapeDtypeStruct((m, n), jnp.float32),
        compiler_params=pltpu.CompilerParams(
            dimension_semantics=("arbitrary",),
        ),
    )(adj, weight)


# speedup vs baseline: 1.6608x; 1.0056x over previous
import functools

import jax
import jax.numpy as jnp
from jax.experimental import pallas as pl
from jax.experimental.pallas import tpu as pltpu


def _mm_body(adj_ref, w_ref, out_ref):
    out_ref[...] = jnp.dot(
        adj_ref[...], w_ref[...], preferred_element_type=jnp.float32
    )


def kernel(adj, weight):
    m, k = adj.shape
    k2, n = weight.shape
    assert k == k2
    bm = 256
    grid = (m // bm,)
    return pl.pallas_call(
        _mm_body,
        grid=grid,
        in_specs=[
            pl.BlockSpec((bm, k), lambda i: (i, 0)),
            pl.BlockSpec((k2, n), lambda i: (0, 0)),
        ],
        out_specs=pl.BlockSpec((bm, n), lambda i: (i, 0)),
        out_shape=jax.ShapeDtypeStruct((m, n), jnp.float32),
        compiler_params=pltpu.CompilerParams(
            dimension_semantics=("arbitrary",),
        ),
    )(adj, weight)
